# Initial kernel scaffold; baseline (speedup 1.0000x reference)
import jax
import jax.numpy as jnp
from jax.experimental import pallas as pl


def kernel(x, z, nodes_mask, edges_mask, mask, params, es):
    raise NotImplementedError("write your pallas kernel here")


# speedup vs baseline: 4.4069x; 4.4069x over previous
import jax
import jax.numpy as jnp
from jax import lax
from jax.experimental import pallas as pl
from jax.experimental.pallas import tpu as pltpu

N = 100
B = 8
D = 4
S_IN = 4
S_TOT = 10
MSG = 32
CATW = D + MSG
TB = 10
NBLK = N // TB
INV_DEG = 1.0 / (N - 1)


def _body(x_ref, zd0_hbm, zd1_hbm,
          w1a0_ref, w1b0_ref, b10_ref, w20_ref, b20_ref,
          w1a1_ref, w1b1_ref, b11_ref, w21_ref, b21_ref,
          ew_ri_ref, eb_ri_ref, ew_ni_ref, eb_ni_ref, ew_nh_ref, eb_nh_ref,
          nw_r_ref, nb_r_ref, nw_i_ref, nb_i_ref,
          nw_ni_ref, nb_ni_ref, nw_nh_ref, nb_nh_ref,
          ow1_ref, ob1_ref, ow2_ref, ob2_ref, ow3_ref, ob3_ref,
          out_ref, he_hbm,
          agg_ref, c0_ref, c1_ref, zb0_ref, zb1_ref, heb_ref, hob_ref,
          sem0, sem1, sem2, sem3):
    f32 = jnp.float32

    w1a0 = w1a0_ref[...]
    w1b0 = w1b0_ref[...]
    w1a1 = w1a1_ref[...]
    w1b1 = w1b1_ref[...]
    w20 = w20_ref[...]
    w21 = w21_ref[...]
    ew_ri = ew_ri_ref[...]
    ew_ni = ew_ni_ref[...]
    ew_nh = ew_nh_ref[...]

    def dot(a, b):
        return jnp.dot(a, b, preferred_element_type=f32)

    x_m = None
    h_node = None

    for s in range(S_TOT):
        if s < S_IN:
            x_in = x_ref[:, :, s * D:(s + 1) * D]
        else:
            x_in = x_m
        xf = x_in.reshape(N * B, D)

        a0 = dot(xf, w1a0).reshape(N, B, MSG)
        a1 = dot(xf, w1a1).reshape(N, B, MSG)
        c0_ref[...] = (dot(xf, w1b0) + b10_ref[...]).reshape(N, B, MSG)
        c1_ref[...] = (dot(xf, w1b1) + b11_ref[...]).reshape(N, B, MSG)

        def blk(b, carry, s=s, a0=a0, a1=a1):
            lo = b * TB
            cz0 = pltpu.make_async_copy(zd0_hbm.at[pl.ds(lo, TB)], zb0_ref, sem0)
            cz1 = pltpu.make_async_copy(zd1_hbm.at[pl.ds(lo, TB)], zb1_ref, sem1)
            cz0.start()
            cz1.start()
            if s > 0:
                che = pltpu.make_async_copy(he_hbm.at[pl.ds(lo, TB)], heb_ref, sem2)
                che.start()
            h1_0 = jax.nn.relu(a0[None, :, :, :] + c0_ref[pl.ds(lo, TB)][:, None, :, :])
            h1_1 = jax.nn.relu(a1[None, :, :, :] + c1_ref[pl.ds(lo, TB)][:, None, :, :])
            h2_0 = jax.nn.relu(dot(h1_0.reshape(TB * N * B, MSG), w20) + b20_ref[...])
            h2_1 = jax.nn.relu(dot(h1_1.reshape(TB * N * B, MSG), w21) + b21_ref[...])
            cz0.wait()
            cz1.wait()
            z0 = zb0_ref[...][..., None]
            z1 = zb1_ref[...][..., None]
            msgs4 = (h2_0.reshape(TB, N, B, MSG) * z0
                     + h2_1.reshape(TB, N, B, MSG) * z1) * 0.5
            if s > 0:
                msgs = msgs4.reshape(TB * N * B, MSG)
                che.wait()
                hid = heb_ref[...].reshape(TB * N * B, MSG)
                g = dot(jnp.concatenate([msgs, hid], axis=1), ew_ri) + eb_ri_ref[...]
                r = jax.nn.sigmoid(g[:, :MSG])
                ig = jax.nn.sigmoid(g[:, MSG:])
                nh = dot(hid, ew_nh) + eb_nh_ref[...]
                n = jnp.tanh(dot(msgs, ew_ni) + eb_ni_ref[...] + r * nh)
                msgs4 = ((1.0 - ig) * n + ig * hid).reshape(TB, N, B, MSG)
            hob_ref[...] = msgs4
            cw = pltpu.make_async_copy(hob_ref, he_hbm.at[pl.ds(lo, TB)], sem3)
            cw.start()
            ii = lax.broadcasted_iota(jnp.int32, (TB, N, 1, 1), 0)
            jj = lax.broadcasted_iota(jnp.int32, (TB, N, 1, 1), 1)
            keep = jnp.where(jj == ii + lo, 0.0, 1.0)
            agg_ref[pl.ds(lo, TB)] = (msgs4 * keep).sum(axis=1) * INV_DEG
            cw.wait()
            return carry

        lax.fori_loop(0, NBLK, blk, 0, unroll=False)

        agg = agg_ref[...]
        cat = jnp.concatenate([x_in, agg], axis=2)
        catf = cat.reshape(N * B, CATW)
        if s > 0:
            hn = h_node.reshape(N * B, CATW)
            gcat = jnp.concatenate([catf, hn], axis=1)
            r = jax.nn.sigmoid(dot(gcat, nw_r_ref[...]) + nb_r_ref[...])
            ig = jax.nn.sigmoid(dot(gcat, nw_i_ref[...]) + nb_i_ref[...])
            nh = dot(hn, nw_nh_ref[...]) + nb_nh_ref[...]
            n = jnp.tanh(dot(catf, nw_ni_ref[...]) + nb_ni_ref[...] + r * nh)
            catf = (1.0 - ig) * n + ig * hn
        h_node = catf.reshape(N, B, CATW)

        d1 = jax.nn.relu(dot(catf, ow1_ref[...]) + ob1_ref[...])
        d2 = jax.nn.relu(dot(d1, ow2_ref[...]) + ob2_ref[...])
        delta = dot(d2, ow3_ref[...]) + ob3_ref[...]
        x_m = x_in + delta.reshape(N, B, D)
        out_ref[:, :, s * D:(s + 1) * D] = x_m


def kernel(x, z, nodes_mask, edges_mask, mask, params, es):
    f32 = jnp.float32
    xt = jnp.transpose(x, (1, 0, 3, 2)).astype(f32).reshape(N, B, S_IN * D)
    zt = jnp.transpose(z, (1, 0, 2)).astype(f32)
    zd = jnp.zeros((N, N, B, 2), f32).at[es[0], es[1]].set(zt)
    zd0 = zd[..., 0]
    zd1 = zd[..., 1]

    def row(v):
        return v.reshape(1, -1).astype(f32)

    m0, m1 = params["msgs"][0], params["msgs"][1]
    ge, gn, o = params["gru_edge"], params["gru_node"], params["out"]

    ew_ri = jnp.concatenate([
        jnp.concatenate([ge["inp"][0][0], ge["inp"][1][0]], axis=1),
        jnp.concatenate([ge["hid"][0][0], ge["hid"][1][0]], axis=1)], axis=0)
    eb_ri = row(jnp.concatenate([ge["inp"][0][1] + ge["hid"][0][1],
                                 ge["inp"][1][1] + ge["hid"][1][1]]))
    nw_r = jnp.concatenate([gn["inp"][0][0], gn["hid"][0][0]], axis=0)
    nw_i = jnp.concatenate([gn["inp"][1][0], gn["hid"][1][0]], axis=0)

    ws = [
        m0["W1"][:D], m0["W1"][D:], row(m0["b1"]), m0["W2"], row(m0["b2"]),
        m1["W1"][:D], m1["W1"][D:], row(m1["b1"]), m1["W2"], row(m1["b2"]),
        ew_ri, eb_ri, ge["inp"][2][0], row(ge["inp"][2][1]),
        ge["hid"][2][0], row(ge["hid"][2][1]),
        nw_r, row(gn["inp"][0][1] + gn["hid"][0][1]),
        nw_i, row(gn["inp"][1][1] + gn["hid"][1][1]),
        gn["inp"][2][0], row(gn["inp"][2][1]),
        gn["hid"][2][0], row(gn["hid"][2][1]),
        o["W1"], row(o["b1"]), o["W2"], row(o["b2"]), o["W3"], row(o["b3"]),
    ]

    vmem = pl.BlockSpec(memory_space=pltpu.MemorySpace.VMEM)
    hbm = pl.BlockSpec(memory_space=pltpu.MemorySpace.HBM)

    out, _ = pl.pallas_call(
        _body,
        out_shape=[
            jax.ShapeDtypeStruct((N, B, S_TOT * D), f32),
            jax.ShapeDtypeStruct((N, N, B, MSG), f32),
        ],
        in_specs=[vmem, hbm, hbm] + [vmem] * 30,
        out_specs=[vmem, hbm],
        scratch_shapes=[
            pltpu.VMEM((N, B, MSG), f32),
            pltpu.VMEM((N, B, MSG), f32),
            pltpu.VMEM((N, B, MSG), f32),
            pltpu.VMEM((TB, N, B), f32),
            pltpu.VMEM((TB, N, B), f32),
            pltpu.VMEM((TB, N, B, MSG), f32),
            pltpu.VMEM((TB, N, B, MSG), f32),
            pltpu.SemaphoreType.DMA,
            pltpu.SemaphoreType.DMA,
            pltpu.SemaphoreType.DMA,
            pltpu.SemaphoreType.DMA,
        ],
    )(xt, zd0, zd1, *ws)
    return out.reshape(N, B, S_TOT, D)


# speedup vs baseline: 4.7728x; 1.0830x over previous
import jax
import jax.numpy as jnp
from jax import lax
from jax.experimental import pallas as pl
from jax.experimental.pallas import tpu as pltpu

N = 100
B = 8
D = 4
S_IN = 4
S_TOT = 10
MSG = 32
CATW = D + MSG
TB = 10
NBLK = N // TB
INV_DEG = 1.0 / (N - 1)


def _body(x_ref, zd0_ref, zd1_ref,
          w1a_ref, w1b_ref, b1_ref, w2_ref, b2_ref,
          ewg_ref, ebg_ref,
          nw_r_ref, nb_r_ref, nw_i_ref, nb_i_ref,
          nw_ni_ref, nb_ni_ref, nw_nh_ref, nb_nh_ref,
          ow1_ref, ob1_ref, ow2_ref, ob2_ref, ow3_ref, ob3_ref,
          out_ref, he_hbm,
          heb0, heb1, hob0, hob1,
          rsem0, rsem1, wsem0, wsem1):
    f32 = jnp.float32

    w1a = w1a_ref[...]
    w1b = w1b_ref[...]
    w2 = w2_ref[...]
    ewg = ewg_ref[...]
    ebg = ebg_ref[...]
    heb = (heb0, heb1)
    hob = (hob0, hob1)
    rsem = (rsem0, rsem1)
    wsem = (wsem0, wsem1)

    def dot(a, b):
        return jnp.dot(a, b, preferred_element_type=f32)

    x_m = None
    h_node = None
    pend_w = []

    for s in range(S_TOT):
        if s < S_IN:
            x_in = x_ref[:, :, s * D:(s + 1) * D]
        else:
            x_in = x_m
        xf = x_in.reshape(N * B, D)

        a01 = dot(xf, w1a).reshape(N, B, 2 * MSG)
        c01 = (dot(xf, w1b) + b1_ref[...]).reshape(N, B, 2 * MSG)

        if s > 0:
            rd0 = pltpu.make_async_copy(he_hbm.at[pl.ds(0, TB)], heb[0], rsem[0])
            rd0.start()
            rd = {0: rd0}

        aggs = []
        for b in range(NBLK):
            lo = b * TB
            h1 = jax.nn.relu(a01[None, :, :, :] + c01[lo:lo + TB][:, None, :, :])
            h2 = jax.nn.relu(dot(h1.reshape(TB * N * B, 2 * MSG), w2) + b2_ref[...])
            h24 = h2.reshape(TB, N, B, 2 * MSG)
            z0 = zd0_ref[lo:lo + TB][..., None]
            z1 = zd1_ref[lo:lo + TB][..., None]
            msgs4 = (h24[..., :MSG] * z0 + h24[..., MSG:] * z1) * 0.5
            if s > 0:
                if b + 1 < NBLK:
                    nxt = pltpu.make_async_copy(
                        he_hbm.at[pl.ds(lo + TB, TB)], heb[(b + 1) % 2], rsem[(b + 1) % 2])
                    nxt.start()
                    rd[b + 1] = nxt
                rd.pop(b).wait()
                hid = heb[b % 2][...].reshape(TB * N * B, MSG)
                msgs = msgs4.reshape(TB * N * B, MSG)
                g = dot(jnp.concatenate([msgs, hid], axis=1), ewg) + ebg
                sg = jax.nn.sigmoid(g[:, :2 * MSG])
                r = sg[:, :MSG]
                ig = sg[:, MSG:]
                n = jnp.tanh(g[:, 2 * MSG:3 * MSG] + r * g[:, 3 * MSG:])
                msgs4 = ((1.0 - ig) * n + ig * hid).reshape(TB, N, B, MSG)
            if len(pend_w) >= 2:
                pend_w.pop(0).wait()
            hob[b % 2][...] = msgs4
            wr = pltpu.make_async_copy(hob[b % 2], he_hbm.at[pl.ds(lo, TB)], wsem[b % 2])
            wr.start()
            pend_w.append(wr)
            ii = lax.broadcasted_iota(jnp.int32, (TB, N, 1, 1), 0)
            jj = lax.broadcasted_iota(jnp.int32, (TB, N, 1, 1), 1)
            keep = jnp.where(jj == ii + lo, 0.0, 1.0)
            aggs.append((msgs4 * keep).sum(axis=1) * INV_DEG)

        while pend_w:
            pend_w.pop(0).wait()

        agg = jnp.concatenate(aggs, axis=0)
        cat = jnp.concatenate([x_in, agg], axis=2)
        catf = cat.reshape(N * B, CATW)
        if s > 0:
            hn = h_node.reshape(N * B, CATW)
            gcat = jnp.concatenate([catf, hn], axis=1)
            r = jax.nn.sigmoid(dot(gcat, nw_r_ref[...]) + nb_r_ref[...])
            ig = jax.nn.sigmoid(dot(gcat, nw_i_ref[...]) + nb_i_ref[...])
            nh = dot(hn, nw_nh_ref[...]) + nb_nh_ref[...]
            n = jnp.tanh(dot(catf, nw_ni_ref[...]) + nb_ni_ref[...] + r * nh)
            catf = (1.0 - ig) * n + ig * hn
        h_node = catf.reshape(N, B, CATW)

        d1 = jax.nn.relu(dot(catf, ow1_ref[...]) + ob1_ref[...])
        d2 = jax.nn.relu(dot(d1, ow2_ref[...]) + ob2_ref[...])
        delta = dot(d2, ow3_ref[...]) + ob3_ref[...]
        x_m = x_in + delta.reshape(N, B, D)
        out_ref[:, :, s * D:(s + 1) * D] = x_m


def kernel(x, z, nodes_mask, edges_mask, mask, params, es):
    f32 = jnp.float32
    xt = jnp.transpose(x, (1, 0, 3, 2)).astype(f32).reshape(N, B, S_IN * D)
    zt = jnp.transpose(z, (1, 0, 2)).astype(f32)
    zd = jnp.zeros((N, N, B, 2), f32).at[es[0], es[1]].set(zt)
    zd0 = zd[..., 0]
    zd1 = zd[..., 1]

    def row(v):
        return v.reshape(1, -1).astype(f32)

    m0, m1 = params["msgs"][0], params["msgs"][1]
    ge, gn, o = params["gru_edge"], params["gru_node"], params["out"]

    zero32 = jnp.zeros((MSG, MSG), f32)
    w1a = jnp.concatenate([m0["W1"][:D], m1["W1"][:D]], axis=1)
    w1b = jnp.concatenate([m0["W1"][D:], m1["W1"][D:]], axis=1)
    b1 = row(jnp.concatenate([m0["b1"], m1["b1"]]))
    w2 = jnp.concatenate([
        jnp.concatenate([m0["W2"], zero32], axis=1),
        jnp.concatenate([zero32, m1["W2"]], axis=1)], axis=0)
    b2 = row(jnp.concatenate([m0["b2"], m1["b2"]]))

    ewg = jnp.concatenate([
        jnp.concatenate([ge["inp"][0][0], ge["inp"][1][0],
                         ge["inp"][2][0], zero32], axis=1),
        jnp.concatenate([ge["hid"][0][0], ge["hid"][1][0],
                         zero32, ge["hid"][2][0]], axis=1)], axis=0)
    ebg = row(jnp.concatenate([
        ge["inp"][0][1] + ge["hid"][0][1],
        ge["inp"][1][1] + ge["hid"][1][1],
        ge["inp"][2][1], ge["hid"][2][1]]))

    nw_r = jnp.concatenate([gn["inp"][0][0], gn["hid"][0][0]], axis=0)
    nw_i = jnp.concatenate([gn["inp"][1][0], gn["hid"][1][0]], axis=0)

    ws = [
        w1a, w1b, b1, w2, b2, ewg, ebg,
        nw_r, row(gn["inp"][0][1] + gn["hid"][0][1]),
        nw_i, row(gn["inp"][1][1] + gn["hid"][1][1]),
        gn["inp"][2][0], row(gn["inp"][2][1]),
        gn["hid"][2][0], row(gn["hid"][2][1]),
        o["W1"], row(o["b1"]), o["W2"], row(o["b2"]), o["W3"], row(o["b3"]),
    ]

    vmem = pl.BlockSpec(memory_space=pltpu.MemorySpace.VMEM)
    hbm = pl.BlockSpec(memory_space=pltpu.MemorySpace.HBM)

    out, _ = pl.pallas_call(
        _body,
        out_shape=[
            jax.ShapeDtypeStruct((N, B, S_TOT * D), f32),
            jax.ShapeDtypeStruct((N, N, B, MSG), f32),
        ],
        in_specs=[vmem] * (3 + len(ws)),
        out_specs=[vmem, hbm],
        scratch_shapes=[
            pltpu.VMEM((TB, N, B, MSG), f32),
            pltpu.VMEM((TB, N, B, MSG), f32),
            pltpu.VMEM((TB, N, B, MSG), f32),
            pltpu.VMEM((TB, N, B, MSG), f32),
            pltpu.SemaphoreType.DMA,
            pltpu.SemaphoreType.DMA,
            pltpu.SemaphoreType.DMA,
            pltpu.SemaphoreType.DMA,
        ],
    )(xt, zd0, zd1, *ws)
    return out.reshape(N, B, S_TOT, D)


# speedup vs baseline: 5.0893x; 1.0663x over previous
import jax
import jax.numpy as jnp
from jax import lax
from jax.experimental import pallas as pl
from jax.experimental.pallas import tpu as pltpu

N = 100
B = 8
D = 4
S_IN = 4
S_TOT = 10
MSG = 32
CATW = D + MSG
TB = 10
NBLK = N // TB
INV_DEG = 1.0 / (N - 1)


def _body(x_ref, zdp_hbm,
          w1a_ref, w1b_ref, b1_ref, w2p_ref, b2p_ref,
          ewg_ref, ebg_ref,
          nw_r_ref, nb_r_ref, nw_i_ref, nb_i_ref,
          nw_ni_ref, nb_ni_ref, nw_nh_ref, nb_nh_ref,
          ow1_ref, ob1_ref, ow2_ref, ob2_ref, ow3_ref, ob3_ref,
          out_ref,
          he_ref, zb0, zb1, zsem0, zsem1):
    f32 = jnp.float32

    w1a = w1a_ref[...]
    w1b = w1b_ref[...]
    w2p = w2p_ref[...]
    ewg = ewg_ref[...]
    ebg = ebg_ref[...]
    zb = (zb0, zb1)
    zsem = (zsem0, zsem1)

    def dot(a, b):
        return jnp.dot(a, b, preferred_element_type=f32)

    x_m = None
    h_node = None

    for s in range(S_TOT):
        if s < S_IN:
            x_in = x_ref[:, :, s * D:(s + 1) * D]
        else:
            x_in = x_m

        xef = x_in[:, :4, :].reshape(N * 4, D)
        xof = x_in[:, 4:, :].reshape(N * 4, D)
        a01 = jnp.concatenate(
            [dot(xef, w1a).reshape(N, 4, 2 * MSG),
             dot(xof, w1a).reshape(N, 4, 2 * MSG)], axis=2)
        c01 = jnp.concatenate(
            [(dot(xef, w1b) + b1_ref[...]).reshape(N, 4, 2 * MSG),
             (dot(xof, w1b) + b1_ref[...]).reshape(N, 4, 2 * MSG)], axis=2)

        zrd0 = pltpu.make_async_copy(zdp_hbm.at[pl.ds(0, TB)], zb[0], zsem[0])
        zrd0.start()
        zrd = {0: zrd0}

        aggs = []
        for b in range(NBLK):
            lo = b * TB
            h1 = jax.nn.relu(a01[None, :, :, :] + c01[lo:lo + TB][:, None, :, :])
            h2 = jax.nn.relu(dot(h1.reshape(TB * N * 4, 4 * MSG), w2p) + b2p_ref[...])
            if b + 1 < NBLK:
                nxt = pltpu.make_async_copy(
                    zdp_hbm.at[pl.ds(lo + TB, TB)], zb[(b + 1) % 2], zsem[(b + 1) % 2])
                nxt.start()
                zrd[b + 1] = nxt
            zrd.pop(b).wait()
            zh = h2.reshape(TB, N, 4, 4 * MSG) * zb[b % 2][...]
            msgs = jnp.concatenate(
                [zh[..., :MSG] + zh[..., MSG:2 * MSG],
                 zh[..., 2 * MSG:3 * MSG] + zh[..., 3 * MSG:]], axis=3)
            if s > 0:
                hid = he_ref[lo:lo + TB]
                g = dot(jnp.concatenate([msgs, hid], axis=3).reshape(TB * N * 4, 4 * MSG),
                        ewg) + ebg
                sg = jax.nn.sigmoid(g[:, :4 * MSG])
                r2 = sg[:, :2 * MSG]
                i2 = sg[:, 2 * MSG:]
                n2 = jnp.tanh(g[:, 4 * MSG:6 * MSG] + r2 * g[:, 6 * MSG:])
                hidf = hid.reshape(TB * N * 4, 2 * MSG)
                msgs = ((1.0 - i2) * n2 + i2 * hidf).reshape(TB, N, 4, 2 * MSG)
            he_ref[lo:lo + TB] = msgs
            ii = lax.broadcasted_iota(jnp.int32, (TB, N, 1, 1), 0)
            jj = lax.broadcasted_iota(jnp.int32, (TB, N, 1, 1), 1)
            keep = jnp.where(jj == ii + lo, 0.0, 1.0)
            aggs.append((msgs * keep).sum(axis=1) * INV_DEG)

        aggp = jnp.concatenate(aggs, axis=0)
        agg = jnp.concatenate([aggp[..., :MSG], aggp[..., MSG:]], axis=1)
        cat = jnp.concatenate([x_in, agg], axis=2)
        catf = cat.reshape(N * B, CATW)
        if s > 0:
            hn = h_node.reshape(N * B, CATW)
            gcat = jnp.concatenate([catf, hn], axis=1)
            r = jax.nn.sigmoid(dot(gcat, nw_r_ref[...]) + nb_r_ref[...])
            ig = jax.nn.sigmoid(dot(gcat, nw_i_ref[...]) + nb_i_ref[...])
            nh = dot(hn, nw_nh_ref[...]) + nb_nh_ref[...]
            n = jnp.tanh(dot(catf, nw_ni_ref[...]) + nb_ni_ref[...] + r * nh)
            catf = (1.0 - ig) * n + ig * hn
        h_node = catf.reshape(N, B, CATW)

        d1 = jax.nn.relu(dot(catf, ow1_ref[...]) + ob1_ref[...])
        d2 = jax.nn.relu(dot(d1, ow2_ref[...]) + ob2_ref[...])
        delta = dot(d2, ow3_ref[...]) + ob3_ref[...]
        x_m = x_in + delta.reshape(N, B, D)
        out_ref[:, :, s * D:(s + 1) * D] = x_m


def kernel(x, z, nodes_mask, edges_mask, mask, params, es):
    f32 = jnp.float32
    xt = jnp.transpose(x, (1, 0, 3, 2)).astype(f32).reshape(N, B, S_IN * D)
    zt = jnp.transpose(z, (1, 0, 2)).astype(f32)
    zd = jnp.zeros((N, N, B, 2), f32).at[es[0], es[1]].set(zt)

    def b32(t):
        return jnp.broadcast_to(t[..., None], t.shape + (MSG,))

    zdp = 0.5 * jnp.concatenate(
        [b32(zd[:, :, :4, 0]), b32(zd[:, :, :4, 1]),
         b32(zd[:, :, 4:, 0]), b32(zd[:, :, 4:, 1])], axis=-1)

    def row(v):
        return v.reshape(1, -1).astype(f32)

    m0, m1 = params["msgs"][0], params["msgs"][1]
    ge, gn, o = params["gru_edge"], params["gru_node"], params["out"]

    zero32 = jnp.zeros((MSG, MSG), f32)
    w1a = jnp.concatenate([m0["W1"][:D], m1["W1"][:D]], axis=1)
    w1b = jnp.concatenate([m0["W1"][D:], m1["W1"][D:]], axis=1)
    b1 = row(jnp.concatenate([m0["b1"], m1["b1"]]))
    w2 = jnp.concatenate([
        jnp.concatenate([m0["W2"], zero32], axis=1),
        jnp.concatenate([zero32, m1["W2"]], axis=1)], axis=0)
    zero64 = jnp.zeros((2 * MSG, 2 * MSG), f32)
    w2p = jnp.concatenate([
        jnp.concatenate([w2, zero64], axis=1),
        jnp.concatenate([zero64, w2], axis=1)], axis=0)
    b2 = jnp.concatenate([m0["b2"], m1["b2"]])
    b2p = row(jnp.concatenate([b2, b2]))

    wir, wii, win = ge["inp"][0][0], ge["inp"][1][0], ge["inp"][2][0]
    whr, whi, whn = ge["hid"][0][0], ge["hid"][1][0], ge["hid"][2][0]
    ewg = jnp.zeros((4 * MSG, 8 * MSG), f32)
    for k, (wi_g, wh_g) in enumerate([(wir, whr), (wii, whi)]):
        ewg = ewg.at[0:MSG, (2 * k) * MSG:(2 * k + 1) * MSG].set(wi_g)
        ewg = ewg.at[MSG:2 * MSG, (2 * k + 1) * MSG:(2 * k + 2) * MSG].set(wi_g)
        ewg = ewg.at[2 * MSG:3 * MSG, (2 * k) * MSG:(2 * k + 1) * MSG].set(wh_g)
        ewg = ewg.at[3 * MSG:4 * MSG, (2 * k + 1) * MSG:(2 * k + 2) * MSG].set(wh_g)
    ewg = ewg.at[0:MSG, 4 * MSG:5 * MSG].set(win)
    ewg = ewg.at[MSG:2 * MSG, 5 * MSG:6 * MSG].set(win)
    ewg = ewg.at[2 * MSG:3 * MSG, 6 * MSG:7 * MSG].set(whn)
    ewg = ewg.at[3 * MSG:4 * MSG, 7 * MSG:8 * MSG].set(whn)
    br = ge["inp"][0][1] + ge["hid"][0][1]
    bi = ge["inp"][1][1] + ge["hid"][1][1]
    ebg = row(jnp.concatenate([br, br, bi, bi,
                               ge["inp"][2][1], ge["inp"][2][1],
                               ge["hid"][2][1], ge["hid"][2][1]]))

    nw_r = jnp.concatenate([gn["inp"][0][0], gn["hid"][0][0]], axis=0)
    nw_i = jnp.concatenate([gn["inp"][1][0], gn["hid"][1][0]], axis=0)

    ws = [
        w1a, w1b, b1, w2p, b2p, ewg, ebg,
        nw_r, row(gn["inp"][0][1] + gn["hid"][0][1]),
        nw_i, row(gn["inp"][1][1] + gn["hid"][1][1]),
        gn["inp"][2][0], row(gn["inp"][2][1]),
        gn["hid"][2][0], row(gn["hid"][2][1]),
        o["W1"], row(o["b1"]), o["W2"], row(o["b2"]), o["W3"], row(o["b3"]),
    ]

    vmem = pl.BlockSpec(memory_space=pltpu.MemorySpace.VMEM)
    hbm = pl.BlockSpec(memory_space=pltpu.MemorySpace.HBM)

    out = pl.pallas_call(
        _body,
        out_shape=jax.ShapeDtypeStruct((N, B, S_TOT * D), f32),
        in_specs=[vmem, hbm] + [vmem] * len(ws),
        out_specs=vmem,
        scratch_shapes=[
            pltpu.VMEM((N, N, 4, 2 * MSG), f32),
            pltpu.VMEM((TB, N, 4, 4 * MSG), f32),
            pltpu.VMEM((TB, N, 4, 4 * MSG), f32),
            pltpu.SemaphoreType.DMA,
            pltpu.SemaphoreType.DMA,
        ],
    )(xt, zdp, *ws)
    return out.reshape(N, B, S_TOT, D)
